# baseline (speedup 1.0000x reference)
import functools
import jax
import jax.numpy as jnp
from jax import lax
from jax.experimental import pallas as pl
from jax.experimental.pallas import tpu as pltpu
from jax.experimental.pallas import tpu_sc as plsc

_N = 10000
_F = 128
_NC = 2
_NS = 16
_NW = _NC * _NS
_CHUNK = 128
_NACC = 10240
_ZROWS = _NACC // _NS
_WB = [(k * _CHUNK, _CHUNK) for k in range(_ZROWS // _CHUNK)]


def _seg_sum(h, src_t, dst_t, q0, q1, qmax):
    mesh = plsc.VectorSubcoreMesh(core_axis_name="c", subcore_axis_name="s")

    @functools.partial(
        pl.kernel,
        out_type=jax.ShapeDtypeStruct((_NC, _NACC, _F), jnp.float32),
        mesh=mesh,
        scratch_types=[
            pltpu.VMEM((qmax, _CHUNK), jnp.int32),
            pltpu.VMEM((qmax, _CHUNK), jnp.int32),
            pltpu.VMEM((_CHUNK, _F), jnp.float32),
            pltpu.VMEM_SHARED((_NACC, _F), jnp.float32),
            pltpu.SemaphoreType.DMA,
        ],
    )
    def seg(h_hbm, src_hbm, dst_hbm, out_hbm, src_v, dst_v, rows, acc, sem):
        c = lax.axis_index("c")
        s = lax.axis_index("s")
        tid = c * _NS + s
        nq = jnp.where(c == 0, q0, q1)

        pltpu.sync_copy(src_hbm.at[tid], src_v)
        pltpu.sync_copy(dst_hbm.at[tid], dst_v)

        @pl.loop(0, _CHUNK)
        def _(i):
            @pl.loop(0, _F, step=16)
            def _(j):
                rows[i, pl.ds(j, 16)] = jnp.zeros((16,), jnp.float32)

        zbase = s * _ZROWS
        for off, sz in _WB:
            pltpu.sync_copy(rows.at[pl.ds(0, sz)], acc.at[pl.ds(zbase + off, sz)])

        plsc.subcore_barrier()

        @pl.loop(0, nq)
        def _(j):
            pltpu.async_copy(h_hbm.at[src_v.at[j]], rows, sem).wait()
            pltpu.sync_copy(rows, acc.at[dst_v.at[j]], add=True)

        plsc.subcore_barrier()

        for off, sz in _WB:
            ob = s * _ZROWS + off
            pltpu.sync_copy(acc.at[pl.ds(ob, sz)], rows.at[pl.ds(0, sz)])
            pltpu.sync_copy(rows.at[pl.ds(0, sz)], out_hbm.at[c].at[pl.ds(ob, sz)])

    return seg(h, src_t, dst_t)


_Q0 = 80
_Q1 = 80


def _mm_relu(p, w):
    BN = 2048

    def body(p_ref, w_ref, o_ref):
        x = p_ref[0] + p_ref[1]
        o_ref[...] = jnp.maximum(
            jnp.dot(x, w_ref[...], preferred_element_type=jnp.float32), 0.0)

    return pl.pallas_call(
        body,
        grid=(_NACC // BN,),
        in_specs=[
            pl.BlockSpec((_NC, BN, _F), lambda i: (0, i, 0)),
            pl.BlockSpec((_F, _F), lambda i: (0, 0)),
        ],
        out_specs=pl.BlockSpec((BN, _F), lambda i: (i, 0)),
        out_shape=jax.ShapeDtypeStruct((_NACC, _F), jnp.float32),
    )(p, w)


def _mm_head(p, w):
    BN = 2048

    def body(p_ref, w_ref, lp_ref, h_ref):
        x = p_ref[0] + p_ref[1]
        h = jnp.dot(x, w_ref[...], preferred_element_type=jnp.float32)
        h_ref[...] = h
        m = jnp.max(h, axis=1, keepdims=True)
        lse = jnp.log(jnp.sum(jnp.exp(h - m), axis=1, keepdims=True)) + m
        lp_ref[...] = h - lse

    return pl.pallas_call(
        body,
        grid=(_NACC // BN,),
        in_specs=[
            pl.BlockSpec((_NC, BN, _F), lambda i: (0, i, 0)),
            pl.BlockSpec((_F, _F), lambda i: (0, 0)),
        ],
        out_specs=[
            pl.BlockSpec((BN, _F), lambda i: (i, 0)),
            pl.BlockSpec((BN, _F), lambda i: (i, 0)),
        ],
        out_shape=[
            jax.ShapeDtypeStruct((_NACC, _F), jnp.float32),
            jax.ShapeDtypeStruct((_NACC, _F), jnp.float32),
        ],
    )(p, w)


def kernel(tinput, adj, W0, W1, W2):
    E = adj.shape[1]
    qmax = max(_Q0, _Q1)
    assert _NS * (_Q0 + _Q1) * _CHUNK >= E

    def to3d(flat):
        a = flat[:_NS * _Q0 * _CHUNK].reshape(_NS, _Q0, _CHUNK)
        b = flat[_NS * _Q0 * _CHUNK:].reshape(_NS, _Q1, _CHUNK)
        a = jnp.pad(a, ((0, 0), (0, qmax - _Q0), (0, 0)))
        b = jnp.pad(b, ((0, 0), (0, qmax - _Q1), (0, 0)))
        return jnp.concatenate([a, b], axis=0)

    pad = _NS * (_Q0 + _Q1) * _CHUNK - E
    src_t = to3d(jnp.concatenate([adj[0], jnp.zeros((pad,), jnp.int32)]))
    dst_t = to3d(jnp.concatenate([adj[1], jnp.full((pad,), _N, jnp.int32)]))

    h = tinput
    for w in (W0, W1):
        p = _seg_sum(h, src_t, dst_t, _Q0, _Q1, qmax)
        h = _mm_relu(p, w)
    p = _seg_sum(h, src_t, dst_t, _Q0, _Q1, qmax)
    lp, h3 = _mm_head(p, W2)
    return (lp[:_N], h3[:_N])


# speedup vs baseline: 1.5372x; 1.5372x over previous
import functools
import jax
import jax.numpy as jnp
from jax import lax
from jax.experimental import pallas as pl
from jax.experimental.pallas import tpu as pltpu
from jax.experimental.pallas import tpu_sc as plsc

_N = 10000
_F = 128
_NC = 2
_NS = 16
_NW = _NC * _NS
_CHUNK = 128
_NACC = 10240
_ZROWS = _NACC // _NS
_WB = [(k * _CHUNK, _CHUNK) for k in range(_ZROWS // _CHUNK)]


def _seg_sum(h, src_t, dst_t, n_chunks):
    mesh = plsc.VectorSubcoreMesh(core_axis_name="c", subcore_axis_name="s")

    @functools.partial(
        pl.kernel,
        out_type=jax.ShapeDtypeStruct((_NC, _NACC, _F), jnp.float32),
        mesh=mesh,
        scratch_types=[
            pltpu.VMEM((n_chunks, _CHUNK), jnp.int32),
            pltpu.VMEM((n_chunks, _CHUNK), jnp.int32),
            pltpu.VMEM((_CHUNK, _F), jnp.float32),
            pltpu.VMEM_SHARED((_NACC, _F), jnp.float32),
            pltpu.SemaphoreType.DMA,
        ],
    )
    def seg(h_hbm, src_hbm, dst_hbm, out_hbm, src_v, dst_v, rows, acc, sem):
        c = lax.axis_index("c")
        s = lax.axis_index("s")
        wid = s * _NC + c

        pltpu.sync_copy(src_hbm.at[wid], src_v)
        pltpu.sync_copy(dst_hbm.at[wid], dst_v)

        @pl.loop(0, _CHUNK)
        def _(i):
            @pl.loop(0, _F, step=16)
            def _(j):
                rows[i, pl.ds(j, 16)] = jnp.zeros((16,), jnp.float32)

        zbase = s * _ZROWS
        for k in range(_ZROWS // _CHUNK):
            pltpu.sync_copy(rows, acc.at[pl.ds(zbase + k * _CHUNK, _CHUNK)])

        plsc.subcore_barrier()

        @pl.loop(0, n_chunks)
        def _(j):
            pltpu.async_copy(h_hbm.at[src_v.at[j]], rows, sem).wait()
            pltpu.sync_copy(rows, acc.at[dst_v.at[j]], add=True)

        plsc.subcore_barrier()

        for k in range(_ZROWS // _CHUNK):
            ob = s * _ZROWS + k * _CHUNK
            pltpu.sync_copy(acc.at[pl.ds(ob, _CHUNK)], rows)
            pltpu.sync_copy(rows, out_hbm.at[c].at[pl.ds(ob, _CHUNK)])

    return seg(h, src_t, dst_t)


def _mm_relu(p, w):
    BN = 2048

    def body(p_ref, w_ref, o_ref):
        x = p_ref[0] + p_ref[1]
        o_ref[...] = jnp.maximum(
            jnp.dot(x, w_ref[...], preferred_element_type=jnp.float32), 0.0)

    return pl.pallas_call(
        body,
        grid=(_NACC // BN,),
        in_specs=[
            pl.BlockSpec((_NC, BN, _F), lambda i: (0, i, 0)),
            pl.BlockSpec((_F, _F), lambda i: (0, 0)),
        ],
        out_specs=pl.BlockSpec((BN, _F), lambda i: (i, 0)),
        out_shape=jax.ShapeDtypeStruct((_NACC, _F), jnp.float32),
    )(p, w)


def _mm_head(p, w):
    BN = 2048

    def body(p_ref, w_ref, lp_ref, h_ref):
        x = p_ref[0] + p_ref[1]
        h = jnp.dot(x, w_ref[...], preferred_element_type=jnp.float32)
        h_ref[...] = h
        m = jnp.max(h, axis=1, keepdims=True)
        lse = jnp.log(jnp.sum(jnp.exp(h - m), axis=1, keepdims=True)) + m
        lp_ref[...] = h - lse

    return pl.pallas_call(
        body,
        grid=(_NACC // BN,),
        in_specs=[
            pl.BlockSpec((_NC, BN, _F), lambda i: (0, i, 0)),
            pl.BlockSpec((_F, _F), lambda i: (0, 0)),
        ],
        out_specs=[
            pl.BlockSpec((BN, _F), lambda i: (i, 0)),
            pl.BlockSpec((BN, _F), lambda i: (i, 0)),
        ],
        out_shape=[
            jax.ShapeDtypeStruct((_NACC, _F), jnp.float32),
            jax.ShapeDtypeStruct((_NACC, _F), jnp.float32),
        ],
    )(p, w)


def kernel(tinput, adj, W0, W1, W2):
    E = adj.shape[1]
    epb = _NW * _CHUNK
    E_pad = ((E + epb - 1) // epb) * epb
    n_chunks = E_pad // epb
    pad = E_pad - E
    src = jnp.concatenate([adj[0], jnp.zeros((pad,), jnp.int32)])
    dst = jnp.concatenate([adj[1], jnp.full((pad,), _N, jnp.int32)])
    src_t = src.reshape(_NW, n_chunks, _CHUNK)
    dst_t = dst.reshape(_NW, n_chunks, _CHUNK)

    h = tinput
    for w in (W0, W1):
        p = _seg_sum(h, src_t, dst_t, n_chunks)
        h = _mm_relu(p, w)
    p = _seg_sum(h, src_t, dst_t, n_chunks)
    lp, h3 = _mm_head(p, W2)
    return (lp[:_N], h3[:_N])


# speedup vs baseline: 1.5380x; 1.0005x over previous
import functools
import jax
import jax.numpy as jnp
from jax import lax
from jax.experimental import pallas as pl
from jax.experimental.pallas import tpu as pltpu
from jax.experimental.pallas import tpu_sc as plsc

_N = 10000
_F = 128
_NC = 2
_NS = 16
_NW = _NC * _NS
_CHUNK = 128
_NACC = 10240
_ZROWS = _NACC // _NS


def _seg_sum(h, src_t, dst_t, n_chunks):
    mesh = plsc.VectorSubcoreMesh(core_axis_name="c", subcore_axis_name="s")

    @functools.partial(
        pl.kernel,
        out_type=jax.ShapeDtypeStruct((_NC, _NACC, _F), jnp.float32),
        mesh=mesh,
        scratch_types=[
            pltpu.VMEM((n_chunks, _CHUNK), jnp.int32),
            pltpu.VMEM((n_chunks, _CHUNK), jnp.int32),
            pltpu.VMEM((_CHUNK, _F), jnp.float32),
            pltpu.VMEM_SHARED((_NACC, _F), jnp.float32),
            pltpu.SemaphoreType.DMA,
        ],
    )
    def seg(h_hbm, src_hbm, dst_hbm, out_hbm, src_v, dst_v, rows, acc, sem):
        c = lax.axis_index("c")
        s = lax.axis_index("s")
        wid = s * _NC + c

        pltpu.sync_copy(src_hbm.at[wid], src_v)
        pltpu.sync_copy(dst_hbm.at[wid], dst_v)

        @pl.loop(0, _CHUNK)
        def _(i):
            @pl.loop(0, _F, step=16)
            def _(j):
                rows[i, pl.ds(j, 16)] = jnp.zeros((16,), jnp.float32)

        zbase = s * _ZROWS
        for k in range(_ZROWS // _CHUNK):
            pltpu.sync_copy(rows, acc.at[pl.ds(zbase + k * _CHUNK, _CHUNK)])

        plsc.subcore_barrier()

        @pl.loop(0, n_chunks)
        def _(j):
            pltpu.async_copy(h_hbm.at[src_v.at[j]], rows, sem).wait()
            pltpu.sync_copy(rows, acc.at[dst_v.at[j]], add=True)

        plsc.subcore_barrier()

        for k in range(_ZROWS // _CHUNK):
            ob = s * _ZROWS + k * _CHUNK
            pltpu.sync_copy(acc.at[pl.ds(ob, _CHUNK)], rows)
            pltpu.sync_copy(rows, out_hbm.at[c].at[pl.ds(ob, _CHUNK)])

    return seg(h, src_t, dst_t)


def _mm_relu(p, w):
    BN = 2048

    def body(p_ref, w_ref, o_ref):
        x = p_ref[0] + p_ref[1]
        o_ref[...] = jnp.maximum(
            jnp.dot(x, w_ref[...], preferred_element_type=jnp.float32), 0.0)

    return pl.pallas_call(
        body,
        grid=(_NACC // BN,),
        in_specs=[
            pl.BlockSpec((_NC, BN, _F), lambda i: (0, i, 0)),
            pl.BlockSpec((_F, _F), lambda i: (0, 0)),
        ],
        out_specs=pl.BlockSpec((BN, _F), lambda i: (i, 0)),
        out_shape=jax.ShapeDtypeStruct((_NACC, _F), jnp.float32),
    )(p, w)


def _mm_head(p, w):
    BN = 2048

    def body(p_ref, w_ref, lp_ref, h_ref):
        x = p_ref[0] + p_ref[1]
        h = jnp.dot(x, w_ref[...], preferred_element_type=jnp.float32)
        h_ref[...] = h
        m = jnp.max(h, axis=1, keepdims=True)
        lse = jnp.log(jnp.sum(jnp.exp(h - m), axis=1, keepdims=True)) + m
        lp_ref[...] = h - lse

    return pl.pallas_call(
        body,
        grid=(_NACC // BN,),
        in_specs=[
            pl.BlockSpec((_NC, BN, _F), lambda i: (0, i, 0)),
            pl.BlockSpec((_F, _F), lambda i: (0, 0)),
        ],
        out_specs=[
            pl.BlockSpec((BN, _F), lambda i: (i, 0)),
            pl.BlockSpec((BN, _F), lambda i: (i, 0)),
        ],
        out_shape=[
            jax.ShapeDtypeStruct((_NACC, _F), jnp.float32),
            jax.ShapeDtypeStruct((_NACC, _F), jnp.float32),
        ],
    )(p, w)


def kernel(tinput, adj, W0, W1, W2):
    E = adj.shape[1]
    epb = _NW * _CHUNK
    E_pad = ((E + epb - 1) // epb) * epb
    n_chunks = E_pad // epb
    pad = E_pad - E
    src = jnp.concatenate([adj[0], jnp.zeros((pad,), jnp.int32)])
    dst = jnp.concatenate([adj[1], jnp.full((pad,), _N, jnp.int32)])
    src_t = src.reshape(_NW, n_chunks, _CHUNK)
    dst_t = dst.reshape(_NW, n_chunks, _CHUNK)

    h = tinput
    for w in (W0, W1):
        p = _seg_sum(h, src_t, dst_t, n_chunks)
        h = _mm_relu(p, w)
    p = _seg_sum(h, src_t, dst_t, n_chunks)
    lp, h3 = _mm_head(p, W2)
    return (lp[:_N], h3[:_N])
